# baseline (speedup 1.0000x reference)
import functools

import jax
import jax.numpy as jnp
from jax import lax
from jax.experimental import pallas as pl
from jax.experimental.pallas import tpu as pltpu
from jax.experimental.pallas import tpu_sc as plsc

_NC = 2
_NS = 16
_NW = _NC * _NS
_CHUNK = 128
_NBUF = 2


@functools.lru_cache(maxsize=None)
def _build_gather(n_chunks: int, d: int):
    b_per_w = n_chunks * _CHUNK
    mesh = plsc.VectorSubcoreMesh(core_axis_name="c", subcore_axis_name="s")

    @functools.partial(
        pl.kernel,
        mesh=mesh,
        compiler_params=pltpu.CompilerParams(use_tc_tiling_on_sc=False),
        out_type=jax.ShapeDtypeStruct((_NW * b_per_w, d), jnp.float32),
        scratch_types=[
            pltpu.VMEM((n_chunks, _CHUNK), jnp.int32),
            pltpu.VMEM((_CHUNK, d), jnp.float32),
            pltpu.VMEM((_CHUNK, d), jnp.float32),
            pltpu.SemaphoreType.DMA,
            pltpu.SemaphoreType.DMA,
        ],
    )
    def k(table_hbm, idx_hbm, out_hbm, idx_v, buf0, buf1, sem0, sem1):
        wid = lax.axis_index("s") * _NC + lax.axis_index("c")
        base = wid * b_per_w
        pltpu.sync_copy(idx_hbm.at[wid], idx_v)

        bufs = (buf0, buf1)
        sems = (sem0, sem1)

        for b in range(_NBUF):
            pltpu.async_copy(table_hbm.at[idx_v.at[b]], bufs[b], sems[b])

        def step(t, carry):
            j = t * _NBUF
            for b in range(_NBUF):
                jj = j + b
                pltpu.make_async_copy(
                    table_hbm.at[pl.ds(0, _CHUNK)], bufs[b], sems[b]
                ).wait()
                pltpu.sync_copy(
                    bufs[b], out_hbm.at[pl.ds(base + jj * _CHUNK, _CHUNK)]
                )
                pltpu.async_copy(
                    table_hbm.at[idx_v.at[jj + _NBUF]], bufs[b], sems[b]
                )
            return carry

        lax.fori_loop(0, n_chunks // _NBUF - 1, step, 0)

        for b in range(_NBUF):
            jj = n_chunks - _NBUF + b
            pltpu.make_async_copy(
                table_hbm.at[pl.ds(0, _CHUNK)], bufs[b], sems[b]
            ).wait()
            pltpu.sync_copy(
                bufs[b], out_hbm.at[pl.ds(base + jj * _CHUNK, _CHUNK)]
            )

    return k


def kernel(indices, embedding):
    d = embedding.shape[1]
    flat = indices.reshape(-1).astype(jnp.int32)
    b = flat.shape[0]
    grain = _NW * _CHUNK
    b_pad = ((b + grain - 1) // grain) * grain
    if b_pad != b:
        flat = jnp.pad(flat, (0, b_pad - b))
    n_chunks = b_pad // grain
    idx3 = flat.reshape(_NW, n_chunks, _CHUNK)
    out = _build_gather(n_chunks, d)(embedding, idx3)
    if b_pad != b:
        out = out[:b]
    return out.reshape(indices.shape + (d,))


# speedup vs baseline: 1.0142x; 1.0142x over previous
import functools

import jax
import jax.numpy as jnp
from jax import lax
from jax.experimental import pallas as pl
from jax.experimental.pallas import tpu as pltpu
from jax.experimental.pallas import tpu_sc as plsc

_NC = 2
_NS = 16
_NW = _NC * _NS
_CHUNK = 128
_KC = 4
_NBUF = 2


@functools.lru_cache(maxsize=None)
def _build_gather(n_chunks: int, d: int):
    b_per_w = n_chunks * _CHUNK
    n_outer = n_chunks // _KC
    mesh = plsc.VectorSubcoreMesh(core_axis_name="c", subcore_axis_name="s")

    @functools.partial(
        pl.kernel,
        mesh=mesh,
        compiler_params=pltpu.CompilerParams(use_tc_tiling_on_sc=False),
        out_type=jax.ShapeDtypeStruct((_NW * n_chunks, _CHUNK, d), jnp.float32),
        scratch_types=[
            pltpu.VMEM((n_chunks, _CHUNK), jnp.int32),
            pltpu.VMEM((_KC, _CHUNK, d), jnp.float32),
            pltpu.VMEM((_KC, _CHUNK, d), jnp.float32),
            pltpu.SemaphoreType.DMA,
            pltpu.SemaphoreType.DMA,
        ],
    )
    def k(table_hbm, idx_hbm, out_hbm, idx_v, buf0, buf1, sem0, sem1):
        wid = lax.axis_index("s") * _NC + lax.axis_index("c")
        base = wid * n_chunks
        pltpu.sync_copy(idx_hbm.at[wid], idx_v)

        bufs = (buf0, buf1)
        sems = (sem0, sem1)

        def fire(jj, b):
            for kk in range(_KC):
                pltpu.async_copy(
                    table_hbm.at[idx_v.at[jj * _KC + kk]],
                    bufs[b].at[kk],
                    sems[b],
                )

        def drain(b):
            for kk in range(_KC):
                pltpu.make_async_copy(
                    table_hbm.at[pl.ds(0, _CHUNK)], bufs[b].at[kk], sems[b]
                ).wait()

        for b in range(_NBUF):
            fire(b, b)

        def step(t, carry):
            j = t * _NBUF
            for b in range(_NBUF):
                jj = j + b
                drain(b)
                pltpu.sync_copy(
                    bufs[b], out_hbm.at[pl.ds(base + jj * _KC, _KC)]
                )
                fire(jj + _NBUF, b)
            return carry

        lax.fori_loop(0, n_outer // _NBUF - 1, step, 0)

        for b in range(_NBUF):
            jj = n_outer - _NBUF + b
            drain(b)
            pltpu.sync_copy(bufs[b], out_hbm.at[pl.ds(base + jj * _KC, _KC)])

    return k


def kernel(indices, embedding):
    d = embedding.shape[1]
    flat = indices.reshape(-1).astype(jnp.int32)
    b = flat.shape[0]
    grain = _NW * _CHUNK * _KC
    b_pad = ((b + grain - 1) // grain) * grain
    if b_pad != b:
        flat = jnp.pad(flat, (0, b_pad - b))
    n_chunks = b_pad // (_NW * _CHUNK)
    idx3 = flat.reshape(_NW, n_chunks, _CHUNK)
    out = _build_gather(n_chunks, d)(embedding, idx3)
    out = out.reshape(b_pad, d)
    if b_pad != b:
        out = out[:b]
    return out.reshape(indices.shape + (d,))
